# Initial kernel scaffold; baseline (speedup 1.0000x reference)
import jax
import jax.numpy as jnp
from jax.experimental import pallas as pl


def kernel(img, pixel_embedding_w, pixel_embedding_b, pos_embedding, final_ln_g, final_ln_b, embedding2pixel_w, embedding2pixel_b, l0_inter_ln_g, l0_inter_ln_b, l0_inter_att_in_w, l0_inter_att_in_b, l0_inter_att_out_w, l0_inter_att_out_b, l0_intra_att_in_w, l0_intra_att_in_b, l0_intra_att_out_w, l0_intra_att_out_b, l0_ff_ln_g, l0_ff_ln_b, l0_ff_w1, l0_ff_b1, l0_ff_w2, l0_ff_b2):
    raise NotImplementedError("write your pallas kernel here")


# speedup vs baseline: 1.5230x; 1.5230x over previous
import functools

import jax
import jax.numpy as jnp
from jax.experimental import pallas as pl
from jax.experimental.pallas import tpu as pltpu

_LN_EPS = 1e-5
_VMEM_LIMIT = 56 * 1024 * 1024
_CENTRE = 4
_HEADS = 8


def _layernorm(x, g, b):
    mu = jnp.mean(x, axis=-1, keepdims=True)
    var = jnp.mean(jnp.square(x - mu), axis=-1, keepdims=True)
    return (x - mu) * jax.lax.rsqrt(var + _LN_EPS) * g + b


def _gelu(x):
    a1, a2, a3, a4, a5 = (0.254829592, -0.284496736, 1.421413741,
                          -1.453152027, 1.061405429)
    pc = 0.3275911
    z = x * 0.7071067811865476
    az = jnp.abs(z)
    t = pl.reciprocal(1.0 + pc * az, approx=True)
    poly = ((((a5 * t + a4) * t + a3) * t + a2) * t + a1) * t
    erf_abs = 1.0 - poly * jnp.exp(-az * az)
    erf = jnp.where(z < 0.0, -erf_abs, erf_abs)
    return 0.5 * x * (1.0 + erf)


def _embed_inter_kernel(px_ref, pos_ref, wpe_ref, bpe_ref, g_ref, b_ref,
                        wq_ref, bq_ref, wkv_ref, bkv_ref, wo_ref, bo_ref,
                        hsel_ref, hselt_ref, o_ref, *, n_nb, centre, heads):
    PN, _ = px_ref[0].shape
    E = wpe_ref.shape[1]
    P = PN // n_nb
    d = E // heads
    scale = 1.0 / (d ** 0.5)

    px = px_ref[0]
    emb = jnp.dot(px.astype(jnp.bfloat16), wpe_ref[...],
                  preferred_element_type=jnp.float32) + bpe_ref[...]
    x3 = emb.astype(jnp.bfloat16).reshape(P, n_nb, E) \
        + pos_ref[...].astype(jnp.bfloat16)[None, :, :]

    xf = x3.astype(jnp.float32).reshape(PN, E)
    xn = _layernorm(xf, g_ref[...], b_ref[...])

    kv = jnp.dot(xn.astype(jnp.bfloat16), wkv_ref[...],
                 preferred_element_type=jnp.float32) + bkv_ref[...]
    xc = xn.reshape(P, n_nb, E)[:, centre, :]
    q = jnp.dot(xc.astype(jnp.bfloat16), wq_ref[...],
                preferred_element_type=jnp.float32) + bq_ref[...]

    kv3 = kv.reshape(P, n_nb, 2 * E)
    k3 = kv3[:, :, :E]
    v3 = kv3[:, :, E:]

    s_all = q[:, None, :] * k3
    s_h = jnp.dot(s_all.reshape(PN, E), hsel_ref[...],
                  preferred_element_type=jnp.float32) * scale
    s_h = s_h.reshape(P, n_nb, heads)

    m = jnp.max(s_h, axis=1, keepdims=True)
    p = jnp.exp(s_h - m)
    den = jnp.sum(p, axis=1, keepdims=True)
    p = p * pl.reciprocal(den, approx=True)

    p_full = jnp.dot(p.reshape(PN, heads), hselt_ref[...],
                     preferred_element_type=jnp.float32).reshape(P, n_nb, E)
    ctx = jnp.sum(p_full * v3, axis=1)

    out = jnp.dot(ctx.astype(jnp.bfloat16), wo_ref[...],
                  preferred_element_type=jnp.float32) + bo_ref[...]
    out = out + xf.reshape(P, n_nb, E)[:, centre, :]
    o_ref[0] = out.astype(jnp.bfloat16)


def _intra_ffn_head_kernel(x_ref, wqkv_ref, bqkv_ref, wo_ref, bo_ref,
                           ffg_ref, ffb_ref, w1_ref, b1_ref, w2_ref, b2_ref,
                           fg_ref, fb_ref, we_ref, be_ref, o_ref,
                           *, heads, d_real):
    x = x_ref[0].astype(jnp.float32)
    P, E = x.shape
    dp = 128
    Hp = heads * dp
    scale = 1.0 / (d_real ** 0.5)

    qkv = jnp.dot(x.astype(jnp.bfloat16), wqkv_ref[...],
                  preferred_element_type=jnp.float32) + bqkv_ref[...]

    ctx = []
    for h in range(heads):
        lo = h * dp
        q_h = qkv[:, lo:lo + dp].astype(jnp.bfloat16)
        k_h = qkv[:, Hp + lo:Hp + lo + dp].astype(jnp.bfloat16)
        v_h = qkv[:, 2 * Hp + lo:2 * Hp + lo + dp].astype(jnp.bfloat16)
        s = jax.lax.dot_general(q_h, k_h, (((1,), (1,)), ((), ())),
                                preferred_element_type=jnp.float32) * scale
        m = jnp.max(s, axis=-1, keepdims=True)
        p = jnp.exp(s - m)
        den = jnp.sum(p, axis=-1, keepdims=True)
        attn = p * pl.reciprocal(den, approx=True)
        ctx.append(jnp.dot(attn.astype(jnp.bfloat16), v_h,
                           preferred_element_type=jnp.float32))
    ctx = jnp.concatenate(ctx, axis=-1)

    att = jnp.dot(ctx.astype(jnp.bfloat16), wo_ref[...],
                  preferred_element_type=jnp.float32) + bo_ref[...]
    y = att + x

    yn = _layernorm(y, ffg_ref[...], ffb_ref[...])
    h1 = jnp.dot(yn.astype(jnp.bfloat16), w1_ref[...],
                 preferred_element_type=jnp.float32) + b1_ref[...]
    h1 = _gelu(h1)
    h2 = jnp.dot(h1.astype(jnp.bfloat16), w2_ref[...],
                 preferred_element_type=jnp.float32) + b2_ref[...]
    z = (h2 + y).astype(jnp.bfloat16).astype(jnp.float32)

    zn = _layernorm(z, fg_ref[...], fb_ref[...])
    out = jnp.dot(zn.astype(jnp.bfloat16), we_ref[...],
                  preferred_element_type=jnp.float32) + be_ref[...]
    o_ref[0] = out


def _pad_heads(mat, heads, d_real, dp=128):
    E = mat.shape[0]
    m = mat.reshape(E, heads, d_real)
    m = jnp.pad(m, ((0, 0), (0, 0), (0, dp - d_real)))
    return m.reshape(E, heads * dp)


def kernel(img, pixel_embedding_w, pixel_embedding_b, pos_embedding,
           final_ln_g, final_ln_b, embedding2pixel_w, embedding2pixel_b,
           l0_inter_ln_g, l0_inter_ln_b, l0_inter_att_in_w, l0_inter_att_in_b,
           l0_inter_att_out_w, l0_inter_att_out_b,
           l0_intra_att_in_w, l0_intra_att_in_b, l0_intra_att_out_w,
           l0_intra_att_out_b, l0_ff_ln_g, l0_ff_ln_b, l0_ff_w1, l0_ff_b1,
           l0_ff_w2, l0_ff_b2):
    B, C, N, Himg, Wimg = img.shape
    P = Himg * Wimg
    E = pos_embedding.shape[-1]
    heads = _HEADS
    d = E // heads
    H = l0_ff_w1.shape[0]

    f32, bf16 = jnp.float32, jnp.bfloat16

    px = img.transpose(0, 3, 4, 2, 1).reshape(B, P * N, C)
    px = jnp.pad(px, ((0, 0), (0, 0), (0, 8 - C)))
    wpe8 = jnp.zeros((8, E), bf16).at[:C].set(pixel_embedding_w.T.astype(bf16))
    bpe = pixel_embedding_b.reshape(1, E).astype(f32)
    pos8 = pos_embedding[0, :N].astype(f32)
    g_in = l0_inter_ln_g.reshape(1, E).astype(f32)
    b_in = l0_inter_ln_b.reshape(1, E).astype(f32)
    wq_t = l0_inter_att_in_w[:E].T.astype(bf16)
    bq = l0_inter_att_in_b[:E].reshape(1, E).astype(f32)
    wkv_t = l0_inter_att_in_w[E:].T.astype(bf16)
    bkv = l0_inter_att_in_b[E:].reshape(1, 2 * E).astype(f32)
    wo_t = l0_inter_att_out_w.T.astype(bf16)
    bo = l0_inter_att_out_b.reshape(1, E).astype(f32)
    hsel = (jnp.arange(E)[:, None] // d == jnp.arange(heads)[None, :]).astype(f32)
    hselt = hsel.T

    kern_a = functools.partial(_embed_inter_kernel, n_nb=N, centre=_CENTRE,
                               heads=heads)
    centre_rows = pl.pallas_call(
        kern_a,
        out_shape=jax.ShapeDtypeStruct((B, P, E), bf16),
        grid_spec=pltpu.PrefetchScalarGridSpec(
            num_scalar_prefetch=0,
            grid=(B,),
            in_specs=[
                pl.BlockSpec((1, P * N, 8), lambda b: (b, 0, 0)),
                pl.BlockSpec((N, E), lambda b: (0, 0)),
                pl.BlockSpec((8, E), lambda b: (0, 0)),
                pl.BlockSpec((1, E), lambda b: (0, 0)),
                pl.BlockSpec((1, E), lambda b: (0, 0)),
                pl.BlockSpec((1, E), lambda b: (0, 0)),
                pl.BlockSpec((E, E), lambda b: (0, 0)),
                pl.BlockSpec((1, E), lambda b: (0, 0)),
                pl.BlockSpec((E, 2 * E), lambda b: (0, 0)),
                pl.BlockSpec((1, 2 * E), lambda b: (0, 0)),
                pl.BlockSpec((E, E), lambda b: (0, 0)),
                pl.BlockSpec((1, E), lambda b: (0, 0)),
                pl.BlockSpec((E, heads), lambda b: (0, 0)),
                pl.BlockSpec((heads, E), lambda b: (0, 0)),
            ],
            out_specs=pl.BlockSpec((1, P, E), lambda b: (b, 0, 0)),
        ),
        compiler_params=pltpu.CompilerParams(
            dimension_semantics=("parallel",),
            vmem_limit_bytes=_VMEM_LIMIT),
    )(px, pos8, wpe8, bpe, g_in, b_in, wq_t, bq, wkv_t, bkv, wo_t, bo,
      hsel, hselt)

    dp = 128
    Hp = heads * dp
    wq_i = _pad_heads(l0_intra_att_in_w[:E].T, heads, d, dp)
    wk_i = _pad_heads(l0_intra_att_in_w[E:2 * E].T, heads, d, dp)
    wv_i = _pad_heads(l0_intra_att_in_w[2 * E:].T, heads, d, dp)
    wqkv_p = jnp.concatenate([wq_i, wk_i, wv_i], axis=1).astype(bf16)
    bq_i = _pad_heads(l0_intra_att_in_b[:E].reshape(1, E), heads, d, dp)
    bk_i = _pad_heads(l0_intra_att_in_b[E:2 * E].reshape(1, E), heads, d, dp)
    bv_i = _pad_heads(l0_intra_att_in_b[2 * E:].reshape(1, E), heads, d, dp)
    bqkv_p = jnp.concatenate([bq_i, bk_i, bv_i], axis=1).astype(f32)
    wo_i = l0_intra_att_out_w.T.reshape(heads, d, E)
    wo_i = jnp.pad(wo_i, ((0, 0), (0, dp - d), (0, 0))).reshape(Hp, E).astype(bf16)
    bo_i = l0_intra_att_out_b.reshape(1, E).astype(f32)
    ffg = l0_ff_ln_g.reshape(1, E).astype(f32)
    ffb = l0_ff_ln_b.reshape(1, E).astype(f32)
    w1_t = l0_ff_w1.T.astype(bf16)
    b1 = l0_ff_b1.reshape(1, H).astype(f32)
    w2_t = l0_ff_w2.T.astype(bf16)
    b2 = l0_ff_b2.reshape(1, E).astype(f32)
    fg = final_ln_g.reshape(1, E).astype(f32)
    fb = final_ln_b.reshape(1, E).astype(f32)
    we = jnp.zeros((E, 128), bf16).at[:, :C].set(embedding2pixel_w.T.astype(bf16))
    be = jnp.zeros((1, 128), f32).at[:, :C].set(
        embedding2pixel_b.reshape(1, C).astype(f32))

    kern_b = functools.partial(_intra_ffn_head_kernel, heads=heads, d_real=d)
    y = pl.pallas_call(
        kern_b,
        out_shape=jax.ShapeDtypeStruct((B, P, 128), f32),
        grid_spec=pltpu.PrefetchScalarGridSpec(
            num_scalar_prefetch=0,
            grid=(B,),
            in_specs=[
                pl.BlockSpec((1, P, E), lambda b: (b, 0, 0)),
                pl.BlockSpec((E, 3 * Hp), lambda b: (0, 0)),
                pl.BlockSpec((1, 3 * Hp), lambda b: (0, 0)),
                pl.BlockSpec((Hp, E), lambda b: (0, 0)),
                pl.BlockSpec((1, E), lambda b: (0, 0)),
                pl.BlockSpec((1, E), lambda b: (0, 0)),
                pl.BlockSpec((1, E), lambda b: (0, 0)),
                pl.BlockSpec((E, H), lambda b: (0, 0)),
                pl.BlockSpec((1, H), lambda b: (0, 0)),
                pl.BlockSpec((H, E), lambda b: (0, 0)),
                pl.BlockSpec((1, E), lambda b: (0, 0)),
                pl.BlockSpec((1, E), lambda b: (0, 0)),
                pl.BlockSpec((1, E), lambda b: (0, 0)),
                pl.BlockSpec((E, 128), lambda b: (0, 0)),
                pl.BlockSpec((1, 128), lambda b: (0, 0)),
            ],
            out_specs=pl.BlockSpec((1, P, 128), lambda b: (b, 0, 0)),
        ),
        compiler_params=pltpu.CompilerParams(
            dimension_semantics=("parallel",),
            vmem_limit_bytes=_VMEM_LIMIT),
    )(centre_rows, wqkv_p, bqkv_p, wo_i, bo_i, ffg, ffb, w1_t, b1, w2_t, b2,
      fg, fb, we, be)

    return y[:, :, :C].reshape(B, Himg, Wimg, C).transpose(0, 3, 1, 2)


# speedup vs baseline: 1.6299x; 1.0702x over previous
import functools

import jax
import jax.numpy as jnp
from jax.experimental import pallas as pl
from jax.experimental.pallas import tpu as pltpu

_LN_EPS = 1e-5
_VMEM_LIMIT = 56 * 1024 * 1024
_CENTRE = 4
_HEADS = 8


def _layernorm(x, g, b):
    mu = jnp.mean(x, axis=-1, keepdims=True)
    var = jnp.mean(jnp.square(x - mu), axis=-1, keepdims=True)
    return (x - mu) * jax.lax.rsqrt(var + _LN_EPS) * g + b


def _gelu(x):
    a1, a2, a3, a4, a5 = (0.254829592, -0.284496736, 1.421413741,
                          -1.453152027, 1.061405429)
    pc = 0.3275911
    z = x * 0.7071067811865476
    az = jnp.abs(z)
    t = pl.reciprocal(1.0 + pc * az, approx=True)
    poly = ((((a5 * t + a4) * t + a3) * t + a2) * t + a1) * t
    erf_abs = 1.0 - poly * jnp.exp(-az * az)
    erf = jnp.where(z < 0.0, -erf_abs, erf_abs)
    return 0.5 * x * (1.0 + erf)


def _embed_inter_kernel(px_ref, pos_ref, wpe_ref, bpe_ref, g_ref, b_ref,
                        wq_ref, bq_ref, wkv_ref, bkv_ref, wo_ref, bo_ref,
                        hsel_ref, hselt_ref, o_ref, *, n_nb, centre, heads):
    PN, _ = px_ref[0].shape
    E = wpe_ref.shape[1]
    P = PN // n_nb
    d = E // heads
    scale = 1.0 / (d ** 0.5)

    px = px_ref[0]
    emb = jnp.dot(px.astype(jnp.bfloat16), wpe_ref[...],
                  preferred_element_type=jnp.float32) + bpe_ref[...]
    x3 = emb.astype(jnp.bfloat16).reshape(n_nb, P, E) \
        + pos_ref[...].astype(jnp.bfloat16)[:, None, :]

    xf = x3.astype(jnp.float32).reshape(PN, E)
    xn = _layernorm(xf, g_ref[...], b_ref[...])

    kv = jnp.dot(xn.astype(jnp.bfloat16), wkv_ref[...],
                 preferred_element_type=jnp.float32) + bkv_ref[...]
    xc = xn.reshape(n_nb, P, E)[centre]
    q = jnp.dot(xc.astype(jnp.bfloat16), wq_ref[...],
                preferred_element_type=jnp.float32) + bq_ref[...]

    kv3 = kv.reshape(n_nb, P, 2 * E)
    k3 = kv3[:, :, :E]
    v3 = kv3[:, :, E:]

    s_all = q[None, :, :] * k3
    s_h = jnp.dot(s_all.reshape(PN, E), hsel_ref[...],
                  preferred_element_type=jnp.float32) * scale
    s_h = s_h.reshape(n_nb, P, heads)

    m = jnp.max(s_h, axis=0, keepdims=True)
    p = jnp.exp(s_h - m)
    den = jnp.sum(p, axis=0, keepdims=True)
    p = p * pl.reciprocal(den, approx=True)

    p_full = jnp.dot(p.reshape(PN, heads), hselt_ref[...],
                     preferred_element_type=jnp.float32).reshape(n_nb, P, E)
    ctx = jnp.sum(p_full * v3, axis=0)

    out = jnp.dot(ctx.astype(jnp.bfloat16), wo_ref[...],
                  preferred_element_type=jnp.float32) + bo_ref[...]
    out = out + xf.reshape(n_nb, P, E)[centre]
    o_ref[0] = out.astype(jnp.bfloat16)


def _intra_ffn_head_kernel(x_ref, wqkv_ref, bqkv_ref, wo_ref, bo_ref,
                           ffg_ref, ffb_ref, w1_ref, b1_ref, w2_ref, b2_ref,
                           fg_ref, fb_ref, we_ref, be_ref, o_ref,
                           *, heads, d_real):
    x = x_ref[0].astype(jnp.float32)
    P, E = x.shape
    dp = 128
    Hp = heads * dp
    scale = 1.0 / (d_real ** 0.5)

    qkv = jnp.dot(x.astype(jnp.bfloat16), wqkv_ref[...],
                  preferred_element_type=jnp.float32) + bqkv_ref[...]

    ctx = []
    for h in range(heads):
        lo = h * dp
        q_h = qkv[:, lo:lo + dp].astype(jnp.bfloat16)
        k_h = qkv[:, Hp + lo:Hp + lo + dp].astype(jnp.bfloat16)
        v_h = qkv[:, 2 * Hp + lo:2 * Hp + lo + dp].astype(jnp.bfloat16)
        s = jax.lax.dot_general(q_h, k_h, (((1,), (1,)), ((), ())),
                                preferred_element_type=jnp.float32) * scale
        m = jnp.max(s, axis=-1, keepdims=True)
        p = jnp.exp(s - m)
        den = jnp.sum(p, axis=-1, keepdims=True)
        attn = p * pl.reciprocal(den, approx=True)
        ctx.append(jnp.dot(attn.astype(jnp.bfloat16), v_h,
                           preferred_element_type=jnp.float32))
    ctx = jnp.concatenate(ctx, axis=-1)

    att = jnp.dot(ctx.astype(jnp.bfloat16), wo_ref[...],
                  preferred_element_type=jnp.float32) + bo_ref[...]
    y = att + x

    yn = _layernorm(y, ffg_ref[...], ffb_ref[...])
    h1 = jnp.dot(yn.astype(jnp.bfloat16), w1_ref[...],
                 preferred_element_type=jnp.float32) + b1_ref[...]
    h1 = _gelu(h1)
    h2 = jnp.dot(h1.astype(jnp.bfloat16), w2_ref[...],
                 preferred_element_type=jnp.float32) + b2_ref[...]
    z = (h2 + y).astype(jnp.bfloat16).astype(jnp.float32)

    zn = _layernorm(z, fg_ref[...], fb_ref[...])
    out = jnp.dot(zn.astype(jnp.bfloat16), we_ref[...],
                  preferred_element_type=jnp.float32) + be_ref[...]
    o_ref[0] = out


def _pad_heads(mat, heads, d_real, dp=128):
    E = mat.shape[0]
    m = mat.reshape(E, heads, d_real)
    m = jnp.pad(m, ((0, 0), (0, 0), (0, dp - d_real)))
    return m.reshape(E, heads * dp)


def kernel(img, pixel_embedding_w, pixel_embedding_b, pos_embedding,
           final_ln_g, final_ln_b, embedding2pixel_w, embedding2pixel_b,
           l0_inter_ln_g, l0_inter_ln_b, l0_inter_att_in_w, l0_inter_att_in_b,
           l0_inter_att_out_w, l0_inter_att_out_b,
           l0_intra_att_in_w, l0_intra_att_in_b, l0_intra_att_out_w,
           l0_intra_att_out_b, l0_ff_ln_g, l0_ff_ln_b, l0_ff_w1, l0_ff_b1,
           l0_ff_w2, l0_ff_b2):
    B, C, N, Himg, Wimg = img.shape
    P = Himg * Wimg
    E = pos_embedding.shape[-1]
    heads = _HEADS
    d = E // heads
    H = l0_ff_w1.shape[0]

    f32, bf16 = jnp.float32, jnp.bfloat16

    px = img.transpose(0, 2, 3, 4, 1).reshape(B, P * N, C)
    px = jnp.pad(px, ((0, 0), (0, 0), (0, 8 - C)))
    wpe8 = jnp.zeros((8, E), bf16).at[:C].set(pixel_embedding_w.T.astype(bf16))
    bpe = pixel_embedding_b.reshape(1, E).astype(f32)
    pos8 = pos_embedding[0, :N].astype(f32)
    g_in = l0_inter_ln_g.reshape(1, E).astype(f32)
    b_in = l0_inter_ln_b.reshape(1, E).astype(f32)
    wq_t = l0_inter_att_in_w[:E].T.astype(bf16)
    bq = l0_inter_att_in_b[:E].reshape(1, E).astype(f32)
    wkv_t = l0_inter_att_in_w[E:].T.astype(bf16)
    bkv = l0_inter_att_in_b[E:].reshape(1, 2 * E).astype(f32)
    wo_t = l0_inter_att_out_w.T.astype(bf16)
    bo = l0_inter_att_out_b.reshape(1, E).astype(f32)
    hsel = (jnp.arange(E)[:, None] // d == jnp.arange(heads)[None, :]).astype(f32)
    hselt = hsel.T

    kern_a = functools.partial(_embed_inter_kernel, n_nb=N, centre=_CENTRE,
                               heads=heads)
    centre_rows = pl.pallas_call(
        kern_a,
        out_shape=jax.ShapeDtypeStruct((B, P, E), bf16),
        grid_spec=pltpu.PrefetchScalarGridSpec(
            num_scalar_prefetch=0,
            grid=(B,),
            in_specs=[
                pl.BlockSpec((1, P * N, 8), lambda b: (b, 0, 0)),
                pl.BlockSpec((N, E), lambda b: (0, 0)),
                pl.BlockSpec((8, E), lambda b: (0, 0)),
                pl.BlockSpec((1, E), lambda b: (0, 0)),
                pl.BlockSpec((1, E), lambda b: (0, 0)),
                pl.BlockSpec((1, E), lambda b: (0, 0)),
                pl.BlockSpec((E, E), lambda b: (0, 0)),
                pl.BlockSpec((1, E), lambda b: (0, 0)),
                pl.BlockSpec((E, 2 * E), lambda b: (0, 0)),
                pl.BlockSpec((1, 2 * E), lambda b: (0, 0)),
                pl.BlockSpec((E, E), lambda b: (0, 0)),
                pl.BlockSpec((1, E), lambda b: (0, 0)),
                pl.BlockSpec((E, heads), lambda b: (0, 0)),
                pl.BlockSpec((heads, E), lambda b: (0, 0)),
            ],
            out_specs=pl.BlockSpec((1, P, E), lambda b: (b, 0, 0)),
        ),
        compiler_params=pltpu.CompilerParams(
            dimension_semantics=("parallel",),
            vmem_limit_bytes=_VMEM_LIMIT),
    )(px, pos8, wpe8, bpe, g_in, b_in, wq_t, bq, wkv_t, bkv, wo_t, bo,
      hsel, hselt)

    dp = 128
    Hp = heads * dp
    wq_i = _pad_heads(l0_intra_att_in_w[:E].T, heads, d, dp)
    wk_i = _pad_heads(l0_intra_att_in_w[E:2 * E].T, heads, d, dp)
    wv_i = _pad_heads(l0_intra_att_in_w[2 * E:].T, heads, d, dp)
    wqkv_p = jnp.concatenate([wq_i, wk_i, wv_i], axis=1).astype(bf16)
    bq_i = _pad_heads(l0_intra_att_in_b[:E].reshape(1, E), heads, d, dp)
    bk_i = _pad_heads(l0_intra_att_in_b[E:2 * E].reshape(1, E), heads, d, dp)
    bv_i = _pad_heads(l0_intra_att_in_b[2 * E:].reshape(1, E), heads, d, dp)
    bqkv_p = jnp.concatenate([bq_i, bk_i, bv_i], axis=1).astype(f32)
    wo_i = l0_intra_att_out_w.T.reshape(heads, d, E)
    wo_i = jnp.pad(wo_i, ((0, 0), (0, dp - d), (0, 0))).reshape(Hp, E).astype(bf16)
    bo_i = l0_intra_att_out_b.reshape(1, E).astype(f32)
    ffg = l0_ff_ln_g.reshape(1, E).astype(f32)
    ffb = l0_ff_ln_b.reshape(1, E).astype(f32)
    w1_t = l0_ff_w1.T.astype(bf16)
    b1 = l0_ff_b1.reshape(1, H).astype(f32)
    w2_t = l0_ff_w2.T.astype(bf16)
    b2 = l0_ff_b2.reshape(1, E).astype(f32)
    fg = final_ln_g.reshape(1, E).astype(f32)
    fb = final_ln_b.reshape(1, E).astype(f32)
    we = jnp.zeros((E, 128), bf16).at[:, :C].set(embedding2pixel_w.T.astype(bf16))
    be = jnp.zeros((1, 128), f32).at[:, :C].set(
        embedding2pixel_b.reshape(1, C).astype(f32))

    kern_b = functools.partial(_intra_ffn_head_kernel, heads=heads, d_real=d)
    y = pl.pallas_call(
        kern_b,
        out_shape=jax.ShapeDtypeStruct((B, P, 128), f32),
        grid_spec=pltpu.PrefetchScalarGridSpec(
            num_scalar_prefetch=0,
            grid=(B,),
            in_specs=[
                pl.BlockSpec((1, P, E), lambda b: (b, 0, 0)),
                pl.BlockSpec((E, 3 * Hp), lambda b: (0, 0)),
                pl.BlockSpec((1, 3 * Hp), lambda b: (0, 0)),
                pl.BlockSpec((Hp, E), lambda b: (0, 0)),
                pl.BlockSpec((1, E), lambda b: (0, 0)),
                pl.BlockSpec((1, E), lambda b: (0, 0)),
                pl.BlockSpec((1, E), lambda b: (0, 0)),
                pl.BlockSpec((E, H), lambda b: (0, 0)),
                pl.BlockSpec((1, H), lambda b: (0, 0)),
                pl.BlockSpec((H, E), lambda b: (0, 0)),
                pl.BlockSpec((1, E), lambda b: (0, 0)),
                pl.BlockSpec((1, E), lambda b: (0, 0)),
                pl.BlockSpec((1, E), lambda b: (0, 0)),
                pl.BlockSpec((E, 128), lambda b: (0, 0)),
                pl.BlockSpec((1, 128), lambda b: (0, 0)),
            ],
            out_specs=pl.BlockSpec((1, P, 128), lambda b: (b, 0, 0)),
        ),
        compiler_params=pltpu.CompilerParams(
            dimension_semantics=("parallel",),
            vmem_limit_bytes=_VMEM_LIMIT),
    )(centre_rows, wqkv_p, bqkv_p, wo_i, bo_i, ffg, ffb, w1_t, b1, w2_t, b2,
      fg, fb, we, be)

    return y[:, :, :C].reshape(B, Himg, Wimg, C).transpose(0, 3, 1, 2)


# speedup vs baseline: 1.6493x; 1.0119x over previous
import functools

import jax
import jax.numpy as jnp
from jax.experimental import pallas as pl
from jax.experimental.pallas import tpu as pltpu

_LN_EPS = 1e-5
_VMEM_LIMIT = 56 * 1024 * 1024
_CENTRE = 4
_HEADS = 8


def _layernorm(x, g, b):
    mu = jnp.mean(x, axis=-1, keepdims=True)
    var = jnp.mean(jnp.square(x - mu), axis=-1, keepdims=True)
    return (x - mu) * jax.lax.rsqrt(var + _LN_EPS) * g + b


def _gelu(x):
    a1, a2, a3, a4, a5 = (0.254829592, -0.284496736, 1.421413741,
                          -1.453152027, 1.061405429)
    pc = 0.3275911
    z = x * 0.7071067811865476
    az = jnp.abs(z)
    t = pl.reciprocal(1.0 + pc * az, approx=True)
    poly = ((((a5 * t + a4) * t + a3) * t + a2) * t + a1) * t
    erf_abs = 1.0 - poly * jnp.exp(-az * az)
    erf = jnp.where(z < 0.0, -erf_abs, erf_abs)
    return 0.5 * x * (1.0 + erf)


def _dot_tb(x, w):
    return jax.lax.dot_general(x, w, (((1,), (1,)), ((), ())),
                               preferred_element_type=jnp.float32)


def _embed_inter_kernel(px_ref, pos_ref, wpe_ref, bpe_ref, g_ref, b_ref,
                        w_in_ref, b_in_ref, wo_ref, bo_ref, o_ref,
                        *, n_nb, centre, heads):
    PN, _ = px_ref[0].shape
    E = wpe_ref.shape[0]
    P = PN // n_nb
    d = E // heads
    scale = 1.0 / (d ** 0.5)

    lane = jax.lax.broadcasted_iota(jnp.int32, (E, heads), 0)
    head = jax.lax.broadcasted_iota(jnp.int32, (E, heads), 1)
    hsel = (lane // d == head).astype(jnp.float32)

    px = px_ref[0]
    emb = _dot_tb(px.astype(jnp.bfloat16), wpe_ref[...]) + bpe_ref[...]
    x3 = emb.astype(jnp.bfloat16).reshape(n_nb, P, E) \
        + pos_ref[0, :n_nb].astype(jnp.bfloat16)[:, None, :]

    xf = x3.astype(jnp.float32).reshape(PN, E)
    xn = _layernorm(xf, g_ref[...], b_ref[...])

    wq = w_in_ref[:E]
    wkv = w_in_ref[E:]
    kv = _dot_tb(xn.astype(jnp.bfloat16), wkv) + b_in_ref[:, E:]
    xc = xn.reshape(n_nb, P, E)[centre]
    q = _dot_tb(xc.astype(jnp.bfloat16), wq) + b_in_ref[:, :E]

    kv3 = kv.reshape(n_nb, P, 2 * E)
    k3 = kv3[:, :, :E]
    v3 = kv3[:, :, E:]

    s_all = q[None, :, :] * k3
    s_h = jnp.dot(s_all.reshape(PN, E), hsel,
                  preferred_element_type=jnp.float32) * scale
    s_h = s_h.reshape(n_nb, P, heads)

    m = jnp.max(s_h, axis=0, keepdims=True)
    p = jnp.exp(s_h - m)
    den = jnp.sum(p, axis=0, keepdims=True)
    p = p * pl.reciprocal(den, approx=True)

    p_full = jax.lax.dot_general(p.reshape(PN, heads), hsel,
                                 (((1,), (1,)), ((), ())),
                                 preferred_element_type=jnp.float32)
    ctx = jnp.sum(p_full.reshape(n_nb, P, E) * v3, axis=0)

    out = _dot_tb(ctx.astype(jnp.bfloat16), wo_ref[...]) + bo_ref[...]
    out = out + xf.reshape(n_nb, P, E)[centre]
    o_ref[0] = out.astype(jnp.bfloat16)


def _intra_ffn_kernel(x_ref, w_in_ref, b_in_ref, wo_ref, bo_ref,
                      ffg_ref, ffb_ref, w1_ref, b1_ref, w2_ref, b2_ref,
                      fg_ref, fb_ref, we_ref, be_ref, o_ref,
                      *, heads):
    x = x_ref[0].astype(jnp.float32)
    P, E = x.shape
    d = E // heads
    scale = 1.0 / (d ** 0.5)

    qkv = _dot_tb(x.astype(jnp.bfloat16), w_in_ref[...]) + b_in_ref[...]

    ctx = []
    for h in range(heads):
        lo = h * d
        q_h = qkv[:, lo:lo + d].astype(jnp.bfloat16)
        k_h = qkv[:, E + lo:E + lo + d].astype(jnp.bfloat16)
        v_h = qkv[:, 2 * E + lo:2 * E + lo + d].astype(jnp.bfloat16)
        s = jax.lax.dot_general(q_h, k_h, (((1,), (1,)), ((), ())),
                                preferred_element_type=jnp.float32) * scale
        m = jnp.max(s, axis=-1, keepdims=True)
        p = jnp.exp(s - m)
        den = jnp.sum(p, axis=-1, keepdims=True)
        attn = p * pl.reciprocal(den, approx=True)
        ctx.append(jnp.dot(attn.astype(jnp.bfloat16), v_h,
                           preferred_element_type=jnp.float32))
    ctx = jnp.concatenate(ctx, axis=-1)

    att = _dot_tb(ctx.astype(jnp.bfloat16), wo_ref[...]) + bo_ref[...]
    y = att + x

    yn = _layernorm(y, ffg_ref[...], ffb_ref[...])
    h1 = _dot_tb(yn.astype(jnp.bfloat16), w1_ref[...]) + b1_ref[...]
    h1 = _gelu(h1)
    h2 = _dot_tb(h1.astype(jnp.bfloat16), w2_ref[...]) + b2_ref[...]
    z = (h2 + y).astype(jnp.bfloat16).astype(jnp.float32)

    zn = _layernorm(z, fg_ref[...], fb_ref[...])
    out_t = jax.lax.dot_general(we_ref[...], zn.astype(jnp.bfloat16),
                                (((1,), (1,)), ((), ())),
                                preferred_element_type=jnp.float32)
    o_ref[0] = out_t + be_ref[...]


def kernel(img, pixel_embedding_w, pixel_embedding_b, pos_embedding,
           final_ln_g, final_ln_b, embedding2pixel_w, embedding2pixel_b,
           l0_inter_ln_g, l0_inter_ln_b, l0_inter_att_in_w, l0_inter_att_in_b,
           l0_inter_att_out_w, l0_inter_att_out_b,
           l0_intra_att_in_w, l0_intra_att_in_b, l0_intra_att_out_w,
           l0_intra_att_out_b, l0_ff_ln_g, l0_ff_ln_b, l0_ff_w1, l0_ff_b1,
           l0_ff_w2, l0_ff_b2):
    B, C, N, Himg, Wimg = img.shape
    P = Himg * Wimg
    E = pos_embedding.shape[-1]
    heads = _HEADS
    H = l0_ff_w1.shape[0]

    f32, bf16 = jnp.float32, jnp.bfloat16

    px = img.transpose(0, 2, 3, 4, 1).reshape(B, N * P, C)
    px = jnp.pad(px, ((0, 0), (0, 0), (0, 8 - C)))

    wpe = jnp.pad(pixel_embedding_w, ((0, 0), (0, 8 - C))).astype(bf16)
    bpe = pixel_embedding_b.reshape(1, E).astype(f32)
    g_in = l0_inter_ln_g.reshape(1, E).astype(f32)
    b_in = l0_inter_ln_b.reshape(1, E).astype(f32)
    w_in_i = l0_inter_att_in_w.astype(bf16)
    b_in_i = l0_inter_att_in_b.reshape(1, 3 * E).astype(f32)
    wo_i = l0_inter_att_out_w.astype(bf16)
    bo_i = l0_inter_att_out_b.reshape(1, E).astype(f32)

    kern_a = functools.partial(_embed_inter_kernel, n_nb=N, centre=_CENTRE,
                               heads=heads)
    centre_rows = pl.pallas_call(
        kern_a,
        out_shape=jax.ShapeDtypeStruct((B, P, E), bf16),
        grid_spec=pltpu.PrefetchScalarGridSpec(
            num_scalar_prefetch=0,
            grid=(B,),
            in_specs=[
                pl.BlockSpec((1, N * P, 8), lambda b: (b, 0, 0)),
                pl.BlockSpec((1, P, E), lambda b: (0, 0, 0)),
                pl.BlockSpec((E, 8), lambda b: (0, 0)),
                pl.BlockSpec((1, E), lambda b: (0, 0)),
                pl.BlockSpec((1, E), lambda b: (0, 0)),
                pl.BlockSpec((1, E), lambda b: (0, 0)),
                pl.BlockSpec((3 * E, E), lambda b: (0, 0)),
                pl.BlockSpec((1, 3 * E), lambda b: (0, 0)),
                pl.BlockSpec((E, E), lambda b: (0, 0)),
                pl.BlockSpec((1, E), lambda b: (0, 0)),
            ],
            out_specs=pl.BlockSpec((1, P, E), lambda b: (b, 0, 0)),
        ),
        compiler_params=pltpu.CompilerParams(
            dimension_semantics=("parallel",),
            vmem_limit_bytes=_VMEM_LIMIT),
    )(px, pos_embedding, wpe, bpe, g_in, b_in, w_in_i, b_in_i, wo_i, bo_i)

    w_in_t = l0_intra_att_in_w.astype(bf16)
    b_in_t = l0_intra_att_in_b.reshape(1, 3 * E).astype(f32)
    wo_t = l0_intra_att_out_w.astype(bf16)
    bo_t = l0_intra_att_out_b.reshape(1, E).astype(f32)
    ffg = l0_ff_ln_g.reshape(1, E).astype(f32)
    ffb = l0_ff_ln_b.reshape(1, E).astype(f32)
    w1 = l0_ff_w1.astype(bf16)
    b1 = l0_ff_b1.reshape(1, H).astype(f32)
    w2 = l0_ff_w2.astype(bf16)
    b2 = l0_ff_b2.reshape(1, E).astype(f32)
    fg = final_ln_g.reshape(1, E).astype(f32)
    fb = final_ln_b.reshape(1, E).astype(f32)
    we = jnp.pad(embedding2pixel_w, ((0, 8 - C), (0, 0))).astype(bf16)
    be = jnp.pad(embedding2pixel_b, (0, 8 - C)).reshape(8, 1).astype(f32)

    kern_b = functools.partial(_intra_ffn_kernel, heads=heads)
    y = pl.pallas_call(
        kern_b,
        out_shape=jax.ShapeDtypeStruct((B, 8, P), f32),
        grid_spec=pltpu.PrefetchScalarGridSpec(
            num_scalar_prefetch=0,
            grid=(B,),
            in_specs=[
                pl.BlockSpec((1, P, E), lambda b: (b, 0, 0)),
                pl.BlockSpec((3 * E, E), lambda b: (0, 0)),
                pl.BlockSpec((1, 3 * E), lambda b: (0, 0)),
                pl.BlockSpec((E, E), lambda b: (0, 0)),
                pl.BlockSpec((1, E), lambda b: (0, 0)),
                pl.BlockSpec((1, E), lambda b: (0, 0)),
                pl.BlockSpec((1, E), lambda b: (0, 0)),
                pl.BlockSpec((H, E), lambda b: (0, 0)),
                pl.BlockSpec((1, H), lambda b: (0, 0)),
                pl.BlockSpec((E, H), lambda b: (0, 0)),
                pl.BlockSpec((1, E), lambda b: (0, 0)),
                pl.BlockSpec((1, E), lambda b: (0, 0)),
                pl.BlockSpec((1, E), lambda b: (0, 0)),
                pl.BlockSpec((8, E), lambda b: (0, 0)),
                pl.BlockSpec((8, 1), lambda b: (0, 0)),
            ],
            out_specs=pl.BlockSpec((1, 8, P), lambda b: (b, 0, 0)),
        ),
        compiler_params=pltpu.CompilerParams(
            dimension_semantics=("parallel",),
            vmem_limit_bytes=_VMEM_LIMIT),
    )(centre_rows, w_in_t, b_in_t, wo_t, bo_t, ffg, ffb, w1, b1, w2, b2,
      fg, fb, we, be)

    return y[:, :C].reshape(B, C, Himg, Wimg)


# speedup vs baseline: 1.7532x; 1.0630x over previous
import functools

import jax
import jax.numpy as jnp
from jax.experimental import pallas as pl
from jax.experimental.pallas import tpu as pltpu

_LN_EPS = 1e-5
_VMEM_LIMIT = 56 * 1024 * 1024
_CENTRE = 4
_HEADS = 8


def _layernorm(x, g, b):
    mu = jnp.mean(x, axis=-1, keepdims=True)
    var = jnp.mean(jnp.square(x - mu), axis=-1, keepdims=True)
    return (x - mu) * jax.lax.rsqrt(var + _LN_EPS) * g + b


def _gelu(x):
    a1, a2, a3, a4, a5 = (0.254829592, -0.284496736, 1.421413741,
                          -1.453152027, 1.061405429)
    pc = 0.3275911
    z = x * 0.7071067811865476
    az = jnp.abs(z)
    t = pl.reciprocal(1.0 + pc * az, approx=True)
    poly = ((((a5 * t + a4) * t + a3) * t + a2) * t + a1) * t
    erf_abs = 1.0 - poly * jnp.exp(-az * az)
    erf = jnp.where(z < 0.0, -erf_abs, erf_abs)
    return 0.5 * x * (1.0 + erf)


def _dot_tb(x, w):
    return jax.lax.dot_general(x, w, (((1,), (1,)), ((), ())),
                               preferred_element_type=jnp.float32)


def _embed_inter_kernel(px_ref, pos_ref, wpe_ref, bpe_ref, g_ref, b_ref,
                        w_in_ref, b_in_ref, wo_ref, bo_ref, o_ref,
                        *, n_nb, centre, heads):
    PN, _ = px_ref[0].shape
    E = wpe_ref.shape[0]
    P = PN // n_nb
    d = E // heads
    scale = 1.0 / (d ** 0.5)

    lane = jax.lax.broadcasted_iota(jnp.int32, (E, heads), 0)
    head = jax.lax.broadcasted_iota(jnp.int32, (E, heads), 1)
    hsel = (lane // d == head).astype(jnp.float32)

    px = px_ref[0]
    emb = _dot_tb(px.astype(jnp.bfloat16),
                  wpe_ref[...].astype(jnp.bfloat16)) + bpe_ref[...]
    x3 = emb.astype(jnp.bfloat16).reshape(n_nb, P, E) \
        + pos_ref[0, :n_nb].astype(jnp.bfloat16)[:, None, :]

    xf = x3.astype(jnp.float32).reshape(PN, E)
    xn = _layernorm(xf, g_ref[...], b_ref[...])

    wq = w_in_ref[:E].astype(jnp.bfloat16)
    wkv = w_in_ref[E:].astype(jnp.bfloat16)
    kv = _dot_tb(xn.astype(jnp.bfloat16), wkv) + b_in_ref[:, E:]
    xc = xn.reshape(n_nb, P, E)[centre]
    q = _dot_tb(xc.astype(jnp.bfloat16), wq) + b_in_ref[:, :E]

    kv3 = kv.reshape(n_nb, P, 2 * E)
    k3 = kv3[:, :, :E]
    v3 = kv3[:, :, E:]

    s_all = q[None, :, :] * k3
    s_h = jnp.dot(s_all.reshape(PN, E), hsel,
                  preferred_element_type=jnp.float32) * scale
    s_h = s_h.reshape(n_nb, P, heads)

    m = jnp.max(s_h, axis=0, keepdims=True)
    p = jnp.exp(s_h - m)
    den = jnp.sum(p, axis=0, keepdims=True)
    p = p * pl.reciprocal(den, approx=True)

    p_full = jax.lax.dot_general(p.reshape(PN, heads), hsel,
                                 (((1,), (1,)), ((), ())),
                                 preferred_element_type=jnp.float32)
    ctx = jnp.sum(p_full.reshape(n_nb, P, E) * v3, axis=0)

    out = _dot_tb(ctx.astype(jnp.bfloat16),
                  wo_ref[...].astype(jnp.bfloat16)) + bo_ref[...]
    out = out + xf.reshape(n_nb, P, E)[centre]
    o_ref[0] = out.astype(jnp.bfloat16)


def _intra_ffn_kernel(x_ref, w_in_ref, b_in_ref, wo_ref, bo_ref,
                      ffg_ref, ffb_ref, w1_ref, b1_ref, w2_ref, b2_ref,
                      fg_ref, fb_ref, we_ref, o_ref,
                      *, heads):
    x = x_ref[0].astype(jnp.float32)
    P, E = x.shape
    d = E // heads
    scale = 1.0 / (d ** 0.5)

    qkv = _dot_tb(x.astype(jnp.bfloat16),
                  w_in_ref[...].astype(jnp.bfloat16)) + b_in_ref[...]

    ctx = []
    for h in range(heads):
        lo = h * d
        q_h = qkv[:, lo:lo + d].astype(jnp.bfloat16)
        k_h = qkv[:, E + lo:E + lo + d].astype(jnp.bfloat16)
        v_h = qkv[:, 2 * E + lo:2 * E + lo + d].astype(jnp.bfloat16)
        s = jax.lax.dot_general(q_h, k_h, (((1,), (1,)), ((), ())),
                                preferred_element_type=jnp.float32) * scale
        m = jnp.max(s, axis=-1, keepdims=True)
        p = jnp.exp(s - m)
        den = jnp.sum(p, axis=-1, keepdims=True)
        attn = p * pl.reciprocal(den, approx=True)
        ctx.append(jnp.dot(attn.astype(jnp.bfloat16), v_h,
                           preferred_element_type=jnp.float32))
    ctx = jnp.concatenate(ctx, axis=-1)

    att = _dot_tb(ctx.astype(jnp.bfloat16),
                  wo_ref[...].astype(jnp.bfloat16)) + bo_ref[...]
    y = att + x

    yn = _layernorm(y, ffg_ref[...], ffb_ref[...])
    h1 = _dot_tb(yn.astype(jnp.bfloat16),
                 w1_ref[...].astype(jnp.bfloat16)) + b1_ref[...]
    h1 = _gelu(h1)
    h2 = _dot_tb(h1.astype(jnp.bfloat16),
                 w2_ref[...].astype(jnp.bfloat16)) + b2_ref[...]
    z = (h2 + y).astype(jnp.bfloat16).astype(jnp.float32)

    zn = _layernorm(z, fg_ref[...], fb_ref[...])
    we8 = jnp.concatenate(
        [we_ref[...].astype(jnp.bfloat16),
         jnp.zeros((8 - we_ref.shape[0], E), jnp.bfloat16)], axis=0)
    out_t = jax.lax.dot_general(we8, zn.astype(jnp.bfloat16),
                                (((1,), (1,)), ((), ())),
                                preferred_element_type=jnp.float32)
    o_ref[0] = out_t


def kernel(img, pixel_embedding_w, pixel_embedding_b, pos_embedding,
           final_ln_g, final_ln_b, embedding2pixel_w, embedding2pixel_b,
           l0_inter_ln_g, l0_inter_ln_b, l0_inter_att_in_w, l0_inter_att_in_b,
           l0_inter_att_out_w, l0_inter_att_out_b,
           l0_intra_att_in_w, l0_intra_att_in_b, l0_intra_att_out_w,
           l0_intra_att_out_b, l0_ff_ln_g, l0_ff_ln_b, l0_ff_w1, l0_ff_b1,
           l0_ff_w2, l0_ff_b2):
    B, C, N, Himg, Wimg = img.shape
    P = Himg * Wimg
    E = pos_embedding.shape[-1]
    heads = _HEADS
    H = l0_ff_w1.shape[0]

    f32, bf16 = jnp.float32, jnp.bfloat16

    px = img.transpose(0, 2, 3, 4, 1).reshape(B, N * P, C)

    bpe = pixel_embedding_b.reshape(1, E)
    g_in = l0_inter_ln_g.reshape(1, E)
    b_in = l0_inter_ln_b.reshape(1, E)
    b_in_i = l0_inter_att_in_b.reshape(1, 3 * E)
    bo_i = l0_inter_att_out_b.reshape(1, E)

    kern_a = functools.partial(_embed_inter_kernel, n_nb=N, centre=_CENTRE,
                               heads=heads)
    centre_rows = pl.pallas_call(
        kern_a,
        out_shape=jax.ShapeDtypeStruct((B, P, E), bf16),
        grid_spec=pltpu.PrefetchScalarGridSpec(
            num_scalar_prefetch=0,
            grid=(B,),
            in_specs=[
                pl.BlockSpec((1, N * P, C), lambda b: (b, 0, 0)),
                pl.BlockSpec((1, P, E), lambda b: (0, 0, 0)),
                pl.BlockSpec((E, C), lambda b: (0, 0)),
                pl.BlockSpec((1, E), lambda b: (0, 0)),
                pl.BlockSpec((1, E), lambda b: (0, 0)),
                pl.BlockSpec((1, E), lambda b: (0, 0)),
                pl.BlockSpec((3 * E, E), lambda b: (0, 0)),
                pl.BlockSpec((1, 3 * E), lambda b: (0, 0)),
                pl.BlockSpec((E, E), lambda b: (0, 0)),
                pl.BlockSpec((1, E), lambda b: (0, 0)),
            ],
            out_specs=pl.BlockSpec((1, P, E), lambda b: (b, 0, 0)),
        ),
        compiler_params=pltpu.CompilerParams(
            dimension_semantics=("parallel",),
            vmem_limit_bytes=_VMEM_LIMIT),
    )(px, pos_embedding, pixel_embedding_w, bpe, g_in, b_in,
      l0_inter_att_in_w, b_in_i, l0_inter_att_out_w, bo_i)

    b_in_t = l0_intra_att_in_b.reshape(1, 3 * E)
    bo_t = l0_intra_att_out_b.reshape(1, E)
    ffg = l0_ff_ln_g.reshape(1, E)
    ffb = l0_ff_ln_b.reshape(1, E)
    b1 = l0_ff_b1.reshape(1, H)
    b2 = l0_ff_b2.reshape(1, E)
    fg = final_ln_g.reshape(1, E)
    fb = final_ln_b.reshape(1, E)

    kern_b = functools.partial(_intra_ffn_kernel, heads=heads)
    y = pl.pallas_call(
        kern_b,
        out_shape=jax.ShapeDtypeStruct((B, 8, P), f32),
        grid_spec=pltpu.PrefetchScalarGridSpec(
            num_scalar_prefetch=0,
            grid=(B,),
            in_specs=[
                pl.BlockSpec((1, P, E), lambda b: (b, 0, 0)),
                pl.BlockSpec((3 * E, E), lambda b: (0, 0)),
                pl.BlockSpec((1, 3 * E), lambda b: (0, 0)),
                pl.BlockSpec((E, E), lambda b: (0, 0)),
                pl.BlockSpec((1, E), lambda b: (0, 0)),
                pl.BlockSpec((1, E), lambda b: (0, 0)),
                pl.BlockSpec((1, E), lambda b: (0, 0)),
                pl.BlockSpec((H, E), lambda b: (0, 0)),
                pl.BlockSpec((1, H), lambda b: (0, 0)),
                pl.BlockSpec((E, H), lambda b: (0, 0)),
                pl.BlockSpec((1, E), lambda b: (0, 0)),
                pl.BlockSpec((1, E), lambda b: (0, 0)),
                pl.BlockSpec((1, E), lambda b: (0, 0)),
                pl.BlockSpec((C, E), lambda b: (0, 0)),
            ],
            out_specs=pl.BlockSpec((1, 8, P), lambda b: (b, 0, 0)),
        ),
        compiler_params=pltpu.CompilerParams(
            dimension_semantics=("parallel",),
            vmem_limit_bytes=_VMEM_LIMIT),
    )(centre_rows, l0_intra_att_in_w, b_in_t, l0_intra_att_out_w, bo_t,
      ffg, ffb, l0_ff_w1, b1, l0_ff_w2, b2, fg, fb, embedding2pixel_w)

    return (y[:, :C] + embedding2pixel_b.reshape(1, C, 1)).reshape(
        B, C, Himg, Wimg)


# speedup vs baseline: 1.8537x; 1.0573x over previous
import functools

import jax
import jax.numpy as jnp
from jax.experimental import pallas as pl
from jax.experimental.pallas import tpu as pltpu

_LN_EPS = 1e-5
_VMEM_LIMIT = 56 * 1024 * 1024
_CENTRE = 4
_HEADS = 8


def _layernorm(x, g, b):
    mu = jnp.mean(x, axis=-1, keepdims=True)
    var = jnp.mean(jnp.square(x - mu), axis=-1, keepdims=True)
    return (x - mu) * jax.lax.rsqrt(var + _LN_EPS) * g + b


def _gelu(x):
    a1, a2, a3, a4, a5 = (0.254829592, -0.284496736, 1.421413741,
                          -1.453152027, 1.061405429)
    pc = 0.3275911
    z = x * 0.7071067811865476
    az = jnp.abs(z)
    t = pl.reciprocal(1.0 + pc * az, approx=True)
    poly = ((((a5 * t + a4) * t + a3) * t + a2) * t + a1) * t
    erf_abs = 1.0 - poly * jnp.exp(-az * az)
    erf = jnp.where(z < 0.0, -erf_abs, erf_abs)
    return 0.5 * x * (1.0 + erf)


def _dot_tb(x, w):
    return jax.lax.dot_general(x, w, (((1,), (1,)), ((), ())),
                               preferred_element_type=jnp.float32)


def _inter_block(px_ref, pos_ref, wpe_ref, bpe_ref, g_ref, b_ref,
                 w_in_ref, b_in_ref, wo_ref, bo_ref,
                 *, n_nb, centre, heads):
    PN, _ = px_ref[0].shape
    E = wpe_ref.shape[0]
    P = PN // n_nb
    d = E // heads
    scale = 1.0 / (d ** 0.5)

    lane = jax.lax.broadcasted_iota(jnp.int32, (E, heads), 0)
    head = jax.lax.broadcasted_iota(jnp.int32, (E, heads), 1)
    hsel = (lane // d == head).astype(jnp.float32)

    px = px_ref[0]
    emb = _dot_tb(px.astype(jnp.bfloat16),
                  wpe_ref[...].astype(jnp.bfloat16)) + bpe_ref[...]
    x3 = emb.astype(jnp.bfloat16).reshape(n_nb, P, E) \
        + pos_ref[0, :n_nb].astype(jnp.bfloat16)[:, None, :]

    xf = x3.astype(jnp.float32).reshape(PN, E)
    xn = _layernorm(xf, g_ref[...], b_ref[...])

    wq = w_in_ref[:E].astype(jnp.bfloat16)
    wkv = w_in_ref[E:].astype(jnp.bfloat16)
    kv = _dot_tb(xn.astype(jnp.bfloat16), wkv) + b_in_ref[:, E:]
    xc = xn.reshape(n_nb, P, E)[centre]
    q = _dot_tb(xc.astype(jnp.bfloat16), wq) + b_in_ref[:, :E]

    kv3 = kv.reshape(n_nb, P, 2 * E)
    k3 = kv3[:, :, :E]
    v3 = kv3[:, :, E:]

    s_all = q[None, :, :] * k3
    s_h = jnp.dot(s_all.reshape(PN, E), hsel,
                  preferred_element_type=jnp.float32) * scale
    s_h = s_h.reshape(n_nb, P, heads)

    m = jnp.max(s_h, axis=0, keepdims=True)
    p = jnp.exp(s_h - m)
    den = jnp.sum(p, axis=0, keepdims=True)
    p = p * pl.reciprocal(den, approx=True)

    p_full = jax.lax.dot_general(p.reshape(PN, heads), hsel,
                                 (((1,), (1,)), ((), ())),
                                 preferred_element_type=jnp.float32)
    ctx = jnp.sum(p_full.reshape(n_nb, P, E) * v3, axis=0)

    out = _dot_tb(ctx.astype(jnp.bfloat16),
                  wo_ref[...].astype(jnp.bfloat16)) + bo_ref[...]
    out = out + xf.reshape(n_nb, P, E)[centre]
    return out.astype(jnp.bfloat16)


def _full_kernel(px_ref, pos_ref, wpe_ref, bpe_ref, g_ref, b_ref,
                 w_in_ref, b_in_ref, wo_ref, bo_ref,
                 w_in2_ref, b_in2_ref, wo2_ref, bo2_ref,
                 ffg_ref, ffb_ref, w1_ref, b1_ref, w2_ref, b2_ref,
                 fg_ref, fb_ref, we_ref, o_ref, *, n_nb, centre, heads):
    centre_rows = _inter_block(px_ref, pos_ref, wpe_ref, bpe_ref, g_ref,
                               b_ref, w_in_ref, b_in_ref, wo_ref, bo_ref,
                               n_nb=n_nb, centre=centre, heads=heads)
    _intra_block(centre_rows, w_in2_ref, b_in2_ref, wo2_ref, bo2_ref,
                 ffg_ref, ffb_ref, w1_ref, b1_ref, w2_ref, b2_ref,
                 fg_ref, fb_ref, we_ref, o_ref, heads=heads)


def _intra_block(x_in, w_in_ref, b_in_ref, wo_ref, bo_ref,
                 ffg_ref, ffb_ref, w1_ref, b1_ref, w2_ref, b2_ref,
                 fg_ref, fb_ref, we_ref, o_ref, *, heads):
    x = x_in.astype(jnp.float32)
    P, E = x.shape
    d = E // heads
    scale = 1.0 / (d ** 0.5)

    qkv = _dot_tb(x.astype(jnp.bfloat16),
                  w_in_ref[...].astype(jnp.bfloat16)) + b_in_ref[...]

    ctx = []
    for h in range(heads):
        lo = h * d
        q_h = qkv[:, lo:lo + d].astype(jnp.bfloat16)
        k_h = qkv[:, E + lo:E + lo + d].astype(jnp.bfloat16)
        v_h = qkv[:, 2 * E + lo:2 * E + lo + d].astype(jnp.bfloat16)
        s = jax.lax.dot_general(q_h, k_h, (((1,), (1,)), ((), ())),
                                preferred_element_type=jnp.float32) * scale
        m = jnp.max(s, axis=-1, keepdims=True)
        p = jnp.exp(s - m)
        den = jnp.sum(p, axis=-1, keepdims=True)
        attn = p * pl.reciprocal(den, approx=True)
        ctx.append(jnp.dot(attn.astype(jnp.bfloat16), v_h,
                           preferred_element_type=jnp.float32))
    ctx = jnp.concatenate(ctx, axis=-1)

    att = _dot_tb(ctx.astype(jnp.bfloat16),
                  wo_ref[...].astype(jnp.bfloat16)) + bo_ref[...]
    y = att + x

    yn = _layernorm(y, ffg_ref[...], ffb_ref[...])
    h1 = _dot_tb(yn.astype(jnp.bfloat16),
                 w1_ref[...].astype(jnp.bfloat16)) + b1_ref[...]
    h1 = _gelu(h1)
    h2 = _dot_tb(h1.astype(jnp.bfloat16),
                 w2_ref[...].astype(jnp.bfloat16)) + b2_ref[...]
    z = (h2 + y).astype(jnp.bfloat16).astype(jnp.float32)

    zn = _layernorm(z, fg_ref[...], fb_ref[...])
    we8 = jnp.concatenate(
        [we_ref[...].astype(jnp.bfloat16),
         jnp.zeros((8 - we_ref.shape[0], E), jnp.bfloat16)], axis=0)
    out_t = jax.lax.dot_general(we8, zn.astype(jnp.bfloat16),
                                (((1,), (1,)), ((), ())),
                                preferred_element_type=jnp.float32)
    o_ref[0] = out_t


def kernel(img, pixel_embedding_w, pixel_embedding_b, pos_embedding,
           final_ln_g, final_ln_b, embedding2pixel_w, embedding2pixel_b,
           l0_inter_ln_g, l0_inter_ln_b, l0_inter_att_in_w, l0_inter_att_in_b,
           l0_inter_att_out_w, l0_inter_att_out_b,
           l0_intra_att_in_w, l0_intra_att_in_b, l0_intra_att_out_w,
           l0_intra_att_out_b, l0_ff_ln_g, l0_ff_ln_b, l0_ff_w1, l0_ff_b1,
           l0_ff_w2, l0_ff_b2):
    B, C, N, Himg, Wimg = img.shape
    P = Himg * Wimg
    E = pos_embedding.shape[-1]
    heads = _HEADS
    H = l0_ff_w1.shape[0]

    f32, bf16 = jnp.float32, jnp.bfloat16

    px = img.transpose(0, 2, 3, 4, 1).reshape(B, N * P, C)

    bpe = pixel_embedding_b.reshape(1, E)
    g_in = l0_inter_ln_g.reshape(1, E)
    b_in = l0_inter_ln_b.reshape(1, E)
    b_in_i = l0_inter_att_in_b.reshape(1, 3 * E)
    bo_i = l0_inter_att_out_b.reshape(1, E)
    b_in_t = l0_intra_att_in_b.reshape(1, 3 * E)
    bo_t = l0_intra_att_out_b.reshape(1, E)
    ffg = l0_ff_ln_g.reshape(1, E)
    ffb = l0_ff_ln_b.reshape(1, E)
    b1 = l0_ff_b1.reshape(1, H)
    b2 = l0_ff_b2.reshape(1, E)
    fg = final_ln_g.reshape(1, E)
    fb = final_ln_b.reshape(1, E)

    _const = lambda b: (0, 0)
    kern = functools.partial(_full_kernel, n_nb=N, centre=_CENTRE,
                             heads=heads)
    y = pl.pallas_call(
        kern,
        out_shape=jax.ShapeDtypeStruct((B, 8, P), f32),
        grid_spec=pltpu.PrefetchScalarGridSpec(
            num_scalar_prefetch=0,
            grid=(B,),
            in_specs=[
                pl.BlockSpec((1, N * P, C), lambda b: (b, 0, 0)),
                pl.BlockSpec((1, P, E), lambda b: (0, 0, 0)),
                pl.BlockSpec((E, C), _const),
                pl.BlockSpec((1, E), _const),
                pl.BlockSpec((1, E), _const),
                pl.BlockSpec((1, E), _const),
                pl.BlockSpec((3 * E, E), _const),
                pl.BlockSpec((1, 3 * E), _const),
                pl.BlockSpec((E, E), _const),
                pl.BlockSpec((1, E), _const),
                pl.BlockSpec((3 * E, E), _const),
                pl.BlockSpec((1, 3 * E), _const),
                pl.BlockSpec((E, E), _const),
                pl.BlockSpec((1, E), _const),
                pl.BlockSpec((1, E), _const),
                pl.BlockSpec((1, E), _const),
                pl.BlockSpec((H, E), _const),
                pl.BlockSpec((1, H), _const),
                pl.BlockSpec((E, H), _const),
                pl.BlockSpec((1, E), _const),
                pl.BlockSpec((1, E), _const),
                pl.BlockSpec((1, E), _const),
                pl.BlockSpec((C, E), _const),
            ],
            out_specs=pl.BlockSpec((1, 8, P), lambda b: (b, 0, 0)),
        ),
        compiler_params=pltpu.CompilerParams(
            dimension_semantics=("parallel",),
            vmem_limit_bytes=_VMEM_LIMIT),
    )(px, pos_embedding, pixel_embedding_w, bpe, g_in, b_in,
      l0_inter_att_in_w, b_in_i, l0_inter_att_out_w, bo_i,
      l0_intra_att_in_w, b_in_t, l0_intra_att_out_w, bo_t,
      ffg, ffb, l0_ff_w1, b1, l0_ff_w2, b2, fg, fb, embedding2pixel_w)

    return (y[:, :C] + embedding2pixel_b.reshape(1, C, 1)).reshape(
        B, C, Himg, Wimg)


# speedup vs baseline: 1.9223x; 1.0370x over previous
import functools

import jax
import jax.numpy as jnp
from jax.experimental import pallas as pl
from jax.experimental.pallas import tpu as pltpu

_LN_EPS = 1e-5
_VMEM_LIMIT = 56 * 1024 * 1024
_CENTRE = 4
_HEADS = 8


def _layernorm(x, g, b):
    mu = jnp.mean(x, axis=-1, keepdims=True)
    var = jnp.mean(jnp.square(x - mu), axis=-1, keepdims=True)
    return (x - mu) * jax.lax.rsqrt(var + _LN_EPS) * g + b


def _gelu(x):
    a1, a2, a3, a4, a5 = (0.254829592, -0.284496736, 1.421413741,
                          -1.453152027, 1.061405429)
    pc = 0.3275911
    z = x * 0.7071067811865476
    az = jnp.abs(z)
    t = pl.reciprocal(1.0 + pc * az, approx=True)
    poly = ((((a5 * t + a4) * t + a3) * t + a2) * t + a1) * t
    erf_abs = 1.0 - poly * jnp.exp(-az * az)
    erf = jnp.where(z < 0.0, -erf_abs, erf_abs)
    return 0.5 * x * (1.0 + erf)


def _dot_tb(x, w):
    return jax.lax.dot_general(x, w, (((1,), (1,)), ((), ())),
                               preferred_element_type=jnp.float32)


def _inter_block(px_ref, pos_ref, wpe_ref, bpe_ref, g_ref, b_ref,
                 w_in_ref, b_in_ref, wo_ref, bo_ref,
                 *, n_nb, centre, heads):
    PN = px_ref.shape[2]
    E = wpe_ref.shape[0]
    P = PN // n_nb
    d = E // heads
    scale = 1.0 / (d ** 0.5)

    lane = jax.lax.broadcasted_iota(jnp.int32, (E, heads), 0)
    head = jax.lax.broadcasted_iota(jnp.int32, (E, heads), 1)
    hsel = (lane // d == head).astype(jnp.float32)

    px = px_ref[0]
    emb = jax.lax.dot_general(px.astype(jnp.bfloat16),
                              wpe_ref[...].astype(jnp.bfloat16),
                              (((0,), (1,)), ((), ())),
                              preferred_element_type=jnp.float32) + bpe_ref[...]
    x3 = emb.astype(jnp.bfloat16).reshape(n_nb, P, E) \
        + pos_ref[0, :n_nb].astype(jnp.bfloat16)[:, None, :]

    xf = x3.astype(jnp.float32).reshape(PN, E)
    xn = _layernorm(xf, g_ref[...], b_ref[...])

    wq = w_in_ref[:E].astype(jnp.bfloat16)
    wkv = w_in_ref[E:].astype(jnp.bfloat16)
    kv = _dot_tb(xn.astype(jnp.bfloat16), wkv) + b_in_ref[:, E:]
    xc = xn.reshape(n_nb, P, E)[centre]
    q = _dot_tb(xc.astype(jnp.bfloat16), wq) + b_in_ref[:, :E]

    kv3 = kv.reshape(n_nb, P, 2 * E)
    k3 = kv3[:, :, :E]
    v3 = kv3[:, :, E:]

    s_all = q[None, :, :] * k3
    s_h = jnp.dot(s_all.reshape(PN, E), hsel,
                  preferred_element_type=jnp.float32) * scale
    s_h = s_h.reshape(n_nb, P, heads)

    m = jnp.max(s_h, axis=0, keepdims=True)
    p = jnp.exp(s_h - m)
    den = jnp.sum(p, axis=0, keepdims=True)
    p = p * pl.reciprocal(den, approx=True)

    p_full = jax.lax.dot_general(p.reshape(PN, heads), hsel,
                                 (((1,), (1,)), ((), ())),
                                 preferred_element_type=jnp.float32)
    ctx = jnp.sum(p_full.reshape(n_nb, P, E) * v3, axis=0)

    out = _dot_tb(ctx.astype(jnp.bfloat16),
                  wo_ref[...].astype(jnp.bfloat16)) + bo_ref[...]
    out = out + xf.reshape(n_nb, P, E)[centre]
    return out.astype(jnp.bfloat16)


def _full_kernel(px_ref, pos_ref, wpe_ref, bpe_ref, g_ref, b_ref,
                 w_in_ref, b_in_ref, wo_ref, bo_ref,
                 w_in2_ref, b_in2_ref, wo2_ref, bo2_ref,
                 ffg_ref, ffb_ref, w1_ref, b1_ref, w2_ref, b2_ref,
                 fg_ref, fb_ref, we_ref, o_ref, *, n_nb, centre, heads):
    centre_rows = _inter_block(px_ref, pos_ref, wpe_ref, bpe_ref, g_ref,
                               b_ref, w_in_ref, b_in_ref, wo_ref, bo_ref,
                               n_nb=n_nb, centre=centre, heads=heads)
    _intra_block(centre_rows, w_in2_ref, b_in2_ref, wo2_ref, bo2_ref,
                 ffg_ref, ffb_ref, w1_ref, b1_ref, w2_ref, b2_ref,
                 fg_ref, fb_ref, we_ref, o_ref, heads=heads)


def _intra_block(x_in, w_in_ref, b_in_ref, wo_ref, bo_ref,
                 ffg_ref, ffb_ref, w1_ref, b1_ref, w2_ref, b2_ref,
                 fg_ref, fb_ref, we_ref, o_ref, *, heads):
    x = x_in.astype(jnp.float32)
    P, E = x.shape
    d = E // heads
    scale = 1.0 / (d ** 0.5)

    qkv = _dot_tb(x.astype(jnp.bfloat16),
                  w_in_ref[...].astype(jnp.bfloat16)) + b_in_ref[...]

    ctx = []
    for h in range(heads):
        lo = h * d
        q_h = qkv[:, lo:lo + d].astype(jnp.bfloat16)
        k_h = qkv[:, E + lo:E + lo + d].astype(jnp.bfloat16)
        v_h = qkv[:, 2 * E + lo:2 * E + lo + d].astype(jnp.bfloat16)
        s = jax.lax.dot_general(q_h, k_h, (((1,), (1,)), ((), ())),
                                preferred_element_type=jnp.float32) * scale
        m = jnp.max(s, axis=-1, keepdims=True)
        p = jnp.exp(s - m)
        den = jnp.sum(p, axis=-1, keepdims=True)
        attn = p * pl.reciprocal(den, approx=True)
        ctx.append(jnp.dot(attn.astype(jnp.bfloat16), v_h,
                           preferred_element_type=jnp.float32))
    ctx = jnp.concatenate(ctx, axis=-1)

    att = _dot_tb(ctx.astype(jnp.bfloat16),
                  wo_ref[...].astype(jnp.bfloat16)) + bo_ref[...]
    y = att + x

    yn = _layernorm(y, ffg_ref[...], ffb_ref[...])
    h1 = _dot_tb(yn.astype(jnp.bfloat16),
                 w1_ref[...].astype(jnp.bfloat16)) + b1_ref[...]
    h1 = _gelu(h1)
    h2 = _dot_tb(h1.astype(jnp.bfloat16),
                 w2_ref[...].astype(jnp.bfloat16)) + b2_ref[...]
    z = (h2 + y).astype(jnp.bfloat16).astype(jnp.float32)

    zn = _layernorm(z, fg_ref[...], fb_ref[...])
    we8 = jnp.concatenate(
        [we_ref[...].astype(jnp.bfloat16),
         jnp.zeros((8 - we_ref.shape[0], E), jnp.bfloat16)], axis=0)
    out_t = jax.lax.dot_general(we8, zn.astype(jnp.bfloat16),
                                (((1,), (1,)), ((), ())),
                                preferred_element_type=jnp.float32)
    o_ref[0] = out_t


def kernel(img, pixel_embedding_w, pixel_embedding_b, pos_embedding,
           final_ln_g, final_ln_b, embedding2pixel_w, embedding2pixel_b,
           l0_inter_ln_g, l0_inter_ln_b, l0_inter_att_in_w, l0_inter_att_in_b,
           l0_inter_att_out_w, l0_inter_att_out_b,
           l0_intra_att_in_w, l0_intra_att_in_b, l0_intra_att_out_w,
           l0_intra_att_out_b, l0_ff_ln_g, l0_ff_ln_b, l0_ff_w1, l0_ff_b1,
           l0_ff_w2, l0_ff_b2):
    B, C, N, Himg, Wimg = img.shape
    P = Himg * Wimg
    E = pos_embedding.shape[-1]
    heads = _HEADS
    H = l0_ff_w1.shape[0]

    f32, bf16 = jnp.float32, jnp.bfloat16

    px = img.reshape(B, C, N * P)

    bpe = pixel_embedding_b.reshape(1, E)
    g_in = l0_inter_ln_g.reshape(1, E)
    b_in = l0_inter_ln_b.reshape(1, E)
    b_in_i = l0_inter_att_in_b.reshape(1, 3 * E)
    bo_i = l0_inter_att_out_b.reshape(1, E)
    b_in_t = l0_intra_att_in_b.reshape(1, 3 * E)
    bo_t = l0_intra_att_out_b.reshape(1, E)
    ffg = l0_ff_ln_g.reshape(1, E)
    ffb = l0_ff_ln_b.reshape(1, E)
    b1 = l0_ff_b1.reshape(1, H)
    b2 = l0_ff_b2.reshape(1, E)
    fg = final_ln_g.reshape(1, E)
    fb = final_ln_b.reshape(1, E)

    _const = lambda b: (0, 0)
    kern = functools.partial(_full_kernel, n_nb=N, centre=_CENTRE,
                             heads=heads)
    y = pl.pallas_call(
        kern,
        out_shape=jax.ShapeDtypeStruct((B, 8, P), f32),
        grid_spec=pltpu.PrefetchScalarGridSpec(
            num_scalar_prefetch=0,
            grid=(B,),
            in_specs=[
                pl.BlockSpec((1, C, N * P), lambda b: (b, 0, 0)),
                pl.BlockSpec((1, P, E), lambda b: (0, 0, 0)),
                pl.BlockSpec((E, C), _const),
                pl.BlockSpec((1, E), _const),
                pl.BlockSpec((1, E), _const),
                pl.BlockSpec((1, E), _const),
                pl.BlockSpec((3 * E, E), _const),
                pl.BlockSpec((1, 3 * E), _const),
                pl.BlockSpec((E, E), _const),
                pl.BlockSpec((1, E), _const),
                pl.BlockSpec((3 * E, E), _const),
                pl.BlockSpec((1, 3 * E), _const),
                pl.BlockSpec((E, E), _const),
                pl.BlockSpec((1, E), _const),
                pl.BlockSpec((1, E), _const),
                pl.BlockSpec((1, E), _const),
                pl.BlockSpec((H, E), _const),
                pl.BlockSpec((1, H), _const),
                pl.BlockSpec((E, H), _const),
                pl.BlockSpec((1, E), _const),
                pl.BlockSpec((1, E), _const),
                pl.BlockSpec((1, E), _const),
                pl.BlockSpec((C, E), _const),
            ],
            out_specs=pl.BlockSpec((1, 8, P), lambda b: (b, 0, 0)),
        ),
        compiler_params=pltpu.CompilerParams(
            dimension_semantics=("parallel",),
            vmem_limit_bytes=_VMEM_LIMIT),
    )(px, pos_embedding, pixel_embedding_w, bpe, g_in, b_in,
      l0_inter_att_in_w, b_in_i, l0_inter_att_out_w, bo_i,
      l0_intra_att_in_w, b_in_t, l0_intra_att_out_w, bo_t,
      ffg, ffb, l0_ff_w1, b1, l0_ff_w2, b2, fg, fb, embedding2pixel_w)

    return (y[:, :C] + embedding2pixel_b.reshape(1, C, 1)).reshape(
        B, C, Himg, Wimg)


# speedup vs baseline: 1.9402x; 1.0093x over previous
import functools

import jax
import jax.numpy as jnp
from jax.experimental import pallas as pl
from jax.experimental.pallas import tpu as pltpu

_LN_EPS = 1e-5
_VMEM_LIMIT = 56 * 1024 * 1024
_CENTRE = 4
_HEADS = 8


def _layernorm(x, g, b):
    mu = jnp.mean(x, axis=-1, keepdims=True)
    var = jnp.mean(jnp.square(x - mu), axis=-1, keepdims=True)
    return (x - mu) * jax.lax.rsqrt(var + _LN_EPS) * g + b


def _gelu(x):
    a1, a2, a3, a4, a5 = (0.254829592, -0.284496736, 1.421413741,
                          -1.453152027, 1.061405429)
    pc = 0.3275911
    z = x * 0.7071067811865476
    az = jnp.abs(z)
    t = pl.reciprocal(1.0 + pc * az, approx=True)
    poly = ((((a5 * t + a4) * t + a3) * t + a2) * t + a1) * t
    erf_abs = 1.0 - poly * jnp.exp(-az * az)
    erf = jnp.where(z < 0.0, -erf_abs, erf_abs)
    return 0.5 * x * (1.0 + erf)


def _dot_tb(x, w):
    return jax.lax.dot_general(x, w, (((1,), (1,)), ((), ())),
                               preferred_element_type=jnp.float32)


def _inter_block(px_ref, pos_ref, wpe_ref, w_in_ref, wo_ref,
                 bpe, g_in, b_ln, bq, bkv, bo,
                 *, n_nb, centre, heads):
    PN = px_ref.shape[2]
    E = wpe_ref.shape[0]
    P = PN // n_nb
    d = E // heads
    scale = 1.0 / (d ** 0.5)

    lane = jax.lax.broadcasted_iota(jnp.int32, (E, heads), 0)
    head = jax.lax.broadcasted_iota(jnp.int32, (E, heads), 1)
    hsel = (lane // d == head).astype(jnp.float32)

    px = px_ref[0]
    emb = jax.lax.dot_general(px.astype(jnp.bfloat16),
                              wpe_ref[...].astype(jnp.bfloat16),
                              (((0,), (1,)), ((), ())),
                              preferred_element_type=jnp.float32) + bpe
    x3 = emb.astype(jnp.bfloat16).reshape(n_nb, P, E) \
        + pos_ref[0, :n_nb].astype(jnp.bfloat16)[:, None, :]

    xf = x3.astype(jnp.float32).reshape(PN, E)
    xn = _layernorm(xf, g_in, b_ln)

    wq = w_in_ref[:E].astype(jnp.bfloat16)
    wkv = w_in_ref[E:].astype(jnp.bfloat16)
    kv = _dot_tb(xn.astype(jnp.bfloat16), wkv) + bkv
    xc = xn.reshape(n_nb, P, E)[centre]
    q = _dot_tb(xc.astype(jnp.bfloat16), wq) + bq

    kv3 = kv.reshape(n_nb, P, 2 * E)
    k3 = kv3[:, :, :E]
    v3 = kv3[:, :, E:]

    s_all = q[None, :, :] * k3
    s_h = jnp.dot(s_all.reshape(PN, E), hsel,
                  preferred_element_type=jnp.float32) * scale
    s_h = s_h.reshape(n_nb, P, heads)

    m = jnp.max(s_h, axis=0, keepdims=True)
    p = jnp.exp(s_h - m)
    den = jnp.sum(p, axis=0, keepdims=True)
    p = p * pl.reciprocal(den, approx=True)

    p_full = jax.lax.dot_general(p.reshape(PN, heads), hsel,
                                 (((1,), (1,)), ((), ())),
                                 preferred_element_type=jnp.float32)
    ctx = jnp.sum(p_full.reshape(n_nb, P, E) * v3, axis=0)

    out = _dot_tb(ctx.astype(jnp.bfloat16),
                  wo_ref[...].astype(jnp.bfloat16)) + bo
    out = out + xf.reshape(n_nb, P, E)[centre]
    return out.astype(jnp.bfloat16)


def _full_kernel(px_ref, pos_ref, wpe_ref, w_in_ref, wo_ref,
                 w_in2_ref, wo2_ref, w1_ref, w2_ref, s_ref, o_ref,
                 *, n_nb, centre, heads):
    s = s_ref[...]
    centre_rows = _inter_block(
        px_ref, pos_ref, wpe_ref, w_in_ref, wo_ref,
        s[0:1], s[1:2], s[2:3], s[3:4],
        jnp.concatenate([s[4:5], s[5:6]], axis=1), s[6:7],
        n_nb=n_nb, centre=centre, heads=heads)
    _intra_block(centre_rows, w_in2_ref, wo2_ref, w1_ref, w2_ref,
                 jnp.concatenate([s[7:8], s[8:9], s[9:10]], axis=1),
                 s[10:11], s[11:12], s[12:13], s[13:14], s[14:15],
                 s[15:16], s[16:17], s[17:20], o_ref, heads=heads)


def _intra_block(x_in, w_in_ref, wo_ref, w1_ref, w2_ref,
                 b_in, bo, ffg, ffb, b1, b2, fg, fb, we, o_ref, *, heads):
    x = x_in.astype(jnp.float32)
    P, E = x.shape
    d = E // heads
    scale = 1.0 / (d ** 0.5)

    qkv = _dot_tb(x.astype(jnp.bfloat16),
                  w_in_ref[...].astype(jnp.bfloat16)) + b_in

    ctx = []
    for h in range(heads):
        lo = h * d
        q_h = qkv[:, lo:lo + d].astype(jnp.bfloat16)
        k_h = qkv[:, E + lo:E + lo + d].astype(jnp.bfloat16)
        v_h = qkv[:, 2 * E + lo:2 * E + lo + d].astype(jnp.bfloat16)
        s = jax.lax.dot_general(q_h, k_h, (((1,), (1,)), ((), ())),
                                preferred_element_type=jnp.float32) * scale
        m = jnp.max(s, axis=-1, keepdims=True)
        p = jnp.exp(s - m)
        den = jnp.sum(p, axis=-1, keepdims=True)
        attn = p * pl.reciprocal(den, approx=True)
        ctx.append(jnp.dot(attn.astype(jnp.bfloat16), v_h,
                           preferred_element_type=jnp.float32))
    ctx = jnp.concatenate(ctx, axis=-1)

    att = _dot_tb(ctx.astype(jnp.bfloat16),
                  wo_ref[...].astype(jnp.bfloat16)) + bo
    y = att + x

    yn = _layernorm(y, ffg, ffb)
    h1 = _dot_tb(yn.astype(jnp.bfloat16),
                 w1_ref[...].astype(jnp.bfloat16)) + b1
    h1 = _gelu(h1)
    h2 = _dot_tb(h1.astype(jnp.bfloat16),
                 w2_ref[...].astype(jnp.bfloat16)) + b2
    z = (h2 + y).astype(jnp.bfloat16).astype(jnp.float32)

    zn = _layernorm(z, fg, fb)
    we8 = jnp.concatenate(
        [we.astype(jnp.bfloat16),
         jnp.zeros((8 - we.shape[0], E), jnp.bfloat16)], axis=0)
    out_t = jax.lax.dot_general(we8, zn.astype(jnp.bfloat16),
                                (((1,), (1,)), ((), ())),
                                preferred_element_type=jnp.float32)
    o_ref[0] = out_t


def kernel(img, pixel_embedding_w, pixel_embedding_b, pos_embedding,
           final_ln_g, final_ln_b, embedding2pixel_w, embedding2pixel_b,
           l0_inter_ln_g, l0_inter_ln_b, l0_inter_att_in_w, l0_inter_att_in_b,
           l0_inter_att_out_w, l0_inter_att_out_b,
           l0_intra_att_in_w, l0_intra_att_in_b, l0_intra_att_out_w,
           l0_intra_att_out_b, l0_ff_ln_g, l0_ff_ln_b, l0_ff_w1, l0_ff_b1,
           l0_ff_w2, l0_ff_b2):
    B, C, N, Himg, Wimg = img.shape
    P = Himg * Wimg
    E = pos_embedding.shape[-1]
    heads = _HEADS
    H = l0_ff_w1.shape[0]

    f32, bf16 = jnp.float32, jnp.bfloat16

    px = img.reshape(B, C, N * P)

    small = jnp.concatenate([
        pixel_embedding_b.reshape(1, E),
        l0_inter_ln_g.reshape(1, E),
        l0_inter_ln_b.reshape(1, E),
        l0_inter_att_in_b.reshape(3, E),
        l0_inter_att_out_b.reshape(1, E),
        l0_intra_att_in_b.reshape(3, E),
        l0_intra_att_out_b.reshape(1, E),
        l0_ff_ln_g.reshape(1, E),
        l0_ff_ln_b.reshape(1, E),
        l0_ff_b1.reshape(1, H),
        l0_ff_b2.reshape(1, E),
        final_ln_g.reshape(1, E),
        final_ln_b.reshape(1, E),
        embedding2pixel_w,
        jnp.zeros((4, E), f32),
    ], axis=0)

    _const = lambda b: (0, 0)
    kern = functools.partial(_full_kernel, n_nb=N, centre=_CENTRE,
                             heads=heads)
    y = pl.pallas_call(
        kern,
        out_shape=jax.ShapeDtypeStruct((B, 8, P), f32),
        grid_spec=pltpu.PrefetchScalarGridSpec(
            num_scalar_prefetch=0,
            grid=(B,),
            in_specs=[
                pl.BlockSpec((1, C, N * P), lambda b: (b, 0, 0)),
                pl.BlockSpec((1, P, E), lambda b: (0, 0, 0)),
                pl.BlockSpec((E, C), _const),
                pl.BlockSpec((3 * E, E), _const),
                pl.BlockSpec((E, E), _const),
                pl.BlockSpec((3 * E, E), _const),
                pl.BlockSpec((E, E), _const),
                pl.BlockSpec((H, E), _const),
                pl.BlockSpec((E, H), _const),
                pl.BlockSpec((24, E), _const),
            ],
            out_specs=pl.BlockSpec((1, 8, P), lambda b: (b, 0, 0)),
        ),
        compiler_params=pltpu.CompilerParams(
            dimension_semantics=("parallel",),
            vmem_limit_bytes=_VMEM_LIMIT),
    )(px, pos_embedding, pixel_embedding_w,
      l0_inter_att_in_w, l0_inter_att_out_w,
      l0_intra_att_in_w, l0_intra_att_out_w,
      l0_ff_w1, l0_ff_w2, small)

    return (y[:, :C] + embedding2pixel_b.reshape(1, C, 1)).reshape(
        B, C, Himg, Wimg)


# speedup vs baseline: 1.9561x; 1.0082x over previous
import functools

import jax
import jax.numpy as jnp
from jax.experimental import pallas as pl
from jax.experimental.pallas import tpu as pltpu

_LN_EPS = 1e-5
_VMEM_LIMIT = 56 * 1024 * 1024
_CENTRE = 4
_HEADS = 8


def _layernorm(x, g, b):
    mu = jnp.mean(x, axis=-1, keepdims=True)
    var = jnp.mean(jnp.square(x - mu), axis=-1, keepdims=True)
    return (x - mu) * jax.lax.rsqrt(var + _LN_EPS) * g + b


def _gelu(x):
    a1, a2, a3, a4, a5 = (0.254829592, -0.284496736, 1.421413741,
                          -1.453152027, 1.061405429)
    pc = 0.3275911
    z = x * 0.7071067811865476
    az = jnp.abs(z)
    t = pl.reciprocal(1.0 + pc * az, approx=True)
    poly = ((((a5 * t + a4) * t + a3) * t + a2) * t + a1) * t
    erf_abs = 1.0 - poly * jnp.exp(-az * az)
    erf = jnp.where(z < 0.0, -erf_abs, erf_abs)
    return 0.5 * x * (1.0 + erf)


def _dot_tb(x, w):
    return jax.lax.dot_general(x, w, (((1,), (1,)), ((), ())),
                               preferred_element_type=jnp.float32)


def _inter_block(px, pos_ref, wpe, wq, wkv, wo,
                 bpe, g_in, b_ln, bq, bkv_b, bo,
                 *, n_nb, centre, heads):
    PN = px.shape[1]
    E = wpe.shape[0]
    P = PN // n_nb
    d = E // heads
    scale = 1.0 / (d ** 0.5)

    lane = jax.lax.broadcasted_iota(jnp.int32, (E, heads), 0)
    head = jax.lax.broadcasted_iota(jnp.int32, (E, heads), 1)
    hsel = (lane // d == head).astype(jnp.float32)

    emb = jax.lax.dot_general(px.astype(jnp.bfloat16), wpe,
                              (((0,), (1,)), ((), ())),
                              preferred_element_type=jnp.float32) + bpe
    x3 = emb.astype(jnp.bfloat16).reshape(n_nb, P, E) \
        + pos_ref[0, :n_nb].astype(jnp.bfloat16)[:, None, :]

    xf = x3.astype(jnp.float32).reshape(PN, E)
    xn = _layernorm(xf, g_in, b_ln)

    kv = _dot_tb(xn.astype(jnp.bfloat16), wkv) + bkv_b
    xc = xn.reshape(n_nb, P, E)[centre]
    q = _dot_tb(xc.astype(jnp.bfloat16), wq) + bq

    kv3 = kv.reshape(n_nb, P, 2 * E)
    k3 = kv3[:, :, :E]
    v3 = kv3[:, :, E:]

    s_all = q[None, :, :] * k3
    s_h = jnp.dot(s_all.reshape(PN, E), hsel,
                  preferred_element_type=jnp.float32) * scale
    s_h = s_h.reshape(n_nb, P, heads)

    m = jnp.max(s_h, axis=0, keepdims=True)
    p = jnp.exp(s_h - m)
    den = jnp.sum(p, axis=0, keepdims=True)
    p = p * pl.reciprocal(den, approx=True)

    p_full = jax.lax.dot_general(p.reshape(PN, heads), hsel,
                                 (((1,), (1,)), ((), ())),
                                 preferred_element_type=jnp.float32)
    ctx = jnp.sum(p_full.reshape(n_nb, P, E) * v3, axis=0)

    out = _dot_tb(ctx.astype(jnp.bfloat16), wo) + bo
    out = out + xf.reshape(n_nb, P, E)[centre]
    return out.astype(jnp.bfloat16)


def _full_kernel(px_ref, pos_ref, wpe_ref, w_in_ref, wo_ref,
                 w_in2_ref, wo2_ref, w1_ref, w2_ref, s_ref, o_ref,
                 *, n_nb, centre, heads, bb):
    bf16 = jnp.bfloat16
    E = wpe_ref.shape[0]
    s = s_ref[...]
    wpe = wpe_ref[...].astype(bf16)
    wq = w_in_ref[:E].astype(bf16)
    wkv = w_in_ref[E:].astype(bf16)
    wo = wo_ref[...].astype(bf16)
    w_in2 = w_in2_ref[...].astype(bf16)
    wo2 = wo2_ref[...].astype(bf16)
    w1 = w1_ref[...].astype(bf16)
    w2 = w2_ref[...].astype(bf16)
    bkv_b = jnp.concatenate([s[4:5], s[5:6]], axis=1)
    b_in2 = jnp.concatenate([s[7:8], s[8:9], s[9:10]], axis=1)
    for i in range(bb):
        centre_rows = _inter_block(
            px_ref[i], pos_ref, wpe, wq, wkv, wo,
            s[0:1], s[1:2], s[2:3], s[3:4], bkv_b, s[6:7],
            n_nb=n_nb, centre=centre, heads=heads)
        _intra_block(centre_rows, w_in2, wo2, w1, w2,
                     b_in2, s[10:11], s[11:12], s[12:13], s[13:14],
                     s[14:15], s[15:16], s[16:17], s[17:20], o_ref, i,
                     heads=heads)


def _intra_block(x_in, w_in2, wo2, w1, w2,
                 b_in, bo, ffg, ffb, b1, b2, fg, fb, we, o_ref, oi, *, heads):
    x = x_in.astype(jnp.float32)
    P, E = x.shape
    d = E // heads
    scale = 1.0 / (d ** 0.5)

    qkv = _dot_tb(x.astype(jnp.bfloat16), w_in2) + b_in

    ctx = []
    for h in range(heads):
        lo = h * d
        q_h = qkv[:, lo:lo + d].astype(jnp.bfloat16)
        k_h = qkv[:, E + lo:E + lo + d].astype(jnp.bfloat16)
        v_h = qkv[:, 2 * E + lo:2 * E + lo + d].astype(jnp.bfloat16)
        s = jax.lax.dot_general(q_h, k_h, (((1,), (1,)), ((), ())),
                                preferred_element_type=jnp.float32) * scale
        m = jnp.max(s, axis=-1, keepdims=True)
        p = jnp.exp(s - m)
        den = jnp.sum(p, axis=-1, keepdims=True)
        attn = p * pl.reciprocal(den, approx=True)
        ctx.append(jnp.dot(attn.astype(jnp.bfloat16), v_h,
                           preferred_element_type=jnp.float32))
    ctx = jnp.concatenate(ctx, axis=-1)

    att = _dot_tb(ctx.astype(jnp.bfloat16), wo2) + bo
    y = att + x

    yn = _layernorm(y, ffg, ffb)
    h1 = _dot_tb(yn.astype(jnp.bfloat16), w1) + b1
    h1 = _gelu(h1)
    h2 = _dot_tb(h1.astype(jnp.bfloat16), w2) + b2
    z = (h2 + y).astype(jnp.bfloat16).astype(jnp.float32)

    zn = _layernorm(z, fg, fb)
    we8 = jnp.concatenate(
        [we.astype(jnp.bfloat16),
         jnp.zeros((8 - we.shape[0], E), jnp.bfloat16)], axis=0)
    out_t = jax.lax.dot_general(we8, zn.astype(jnp.bfloat16),
                                (((1,), (1,)), ((), ())),
                                preferred_element_type=jnp.float32)
    o_ref[oi] = out_t


def kernel(img, pixel_embedding_w, pixel_embedding_b, pos_embedding,
           final_ln_g, final_ln_b, embedding2pixel_w, embedding2pixel_b,
           l0_inter_ln_g, l0_inter_ln_b, l0_inter_att_in_w, l0_inter_att_in_b,
           l0_inter_att_out_w, l0_inter_att_out_b,
           l0_intra_att_in_w, l0_intra_att_in_b, l0_intra_att_out_w,
           l0_intra_att_out_b, l0_ff_ln_g, l0_ff_ln_b, l0_ff_w1, l0_ff_b1,
           l0_ff_w2, l0_ff_b2):
    B, C, N, Himg, Wimg = img.shape
    P = Himg * Wimg
    E = pos_embedding.shape[-1]
    heads = _HEADS
    H = l0_ff_w1.shape[0]

    f32, bf16 = jnp.float32, jnp.bfloat16

    px = img.reshape(B, C, N * P)

    small = jnp.concatenate([
        pixel_embedding_b.reshape(1, E),
        l0_inter_ln_g.reshape(1, E),
        l0_inter_ln_b.reshape(1, E),
        l0_inter_att_in_b.reshape(3, E),
        l0_inter_att_out_b.reshape(1, E),
        l0_intra_att_in_b.reshape(3, E),
        l0_intra_att_out_b.reshape(1, E),
        l0_ff_ln_g.reshape(1, E),
        l0_ff_ln_b.reshape(1, E),
        l0_ff_b1.reshape(1, H),
        l0_ff_b2.reshape(1, E),
        final_ln_g.reshape(1, E),
        final_ln_b.reshape(1, E),
        embedding2pixel_w,
        jnp.zeros((4, E), f32),
    ], axis=0)

    _const = lambda b: (0, 0)
    bb = 2 if B % 2 == 0 else 1
    kern = functools.partial(_full_kernel, n_nb=N, centre=_CENTRE,
                             heads=heads, bb=bb)
    y = pl.pallas_call(
        kern,
        out_shape=jax.ShapeDtypeStruct((B, 8, P), f32),
        grid_spec=pltpu.PrefetchScalarGridSpec(
            num_scalar_prefetch=0,
            grid=(B // bb,),
            in_specs=[
                pl.BlockSpec((bb, C, N * P), lambda b: (b, 0, 0)),
                pl.BlockSpec((1, P, E), lambda b: (0, 0, 0)),
                pl.BlockSpec((E, C), _const),
                pl.BlockSpec((3 * E, E), _const),
                pl.BlockSpec((E, E), _const),
                pl.BlockSpec((3 * E, E), _const),
                pl.BlockSpec((E, E), _const),
                pl.BlockSpec((H, E), _const),
                pl.BlockSpec((E, H), _const),
                pl.BlockSpec((24, E), _const),
            ],
            out_specs=pl.BlockSpec((bb, 8, P), lambda b: (b, 0, 0)),
        ),
        compiler_params=pltpu.CompilerParams(
            dimension_semantics=("parallel",),
            vmem_limit_bytes=_VMEM_LIMIT),
    )(px, pos_embedding, pixel_embedding_w,
      l0_inter_att_in_w, l0_inter_att_out_w,
      l0_intra_att_in_w, l0_intra_att_out_w,
      l0_ff_w1, l0_ff_w2, small)

    return (y[:, :C] + embedding2pixel_b.reshape(1, C, 1)).reshape(
        B, C, Himg, Wimg)
